# Initial kernel scaffold; baseline (speedup 1.0000x reference)
import functools

import jax
import jax.numpy as jnp
from jax.experimental import pallas as pl

_IDX_BITS = 12
_IDX_MASK = (1 << _IDX_BITS) - 1
_KEY_MASK = ~_IDX_MASK
_INF_KEY = jnp.int32(0x7FFFFFFF)


def _pass_a(xyz1_ref, xyz2_ref, p2_ref, p1_ref, w1_ref, b1_ref,
            y1_ref, st_ref, *, m, tn):
    @pl.when(jnp.logical_and(pl.program_id(0) == 0, pl.program_id(1) == 0))
    def _init():
        st_ref[...] = jnp.zeros_like(st_ref)

    x = xyz1_ref[0]
    y = xyz2_ref[0]
    d2 = ((y[:, 0:1] - x[0:1, :]) ** 2
          + (y[:, 1:2] - x[1:2, :]) ** 2
          + (y[:, 2:3] - x[2:3, :]) ** 2)

    iota = jax.lax.broadcasted_iota(jnp.int32, (m, tn), 0)
    bits = jax.lax.bitcast_convert_type(d2, jnp.int32)
    key = (bits & _KEY_MASK) | iota

    sel = []
    for _ in range(3):
        mn = jnp.min(key, axis=0, keepdims=True)
        sel.append(mn)
        key = jnp.where(key == mn, _INF_KEY, key)

    idx = [s & _IDX_MASK for s in sel]
    dst = [jax.lax.bitcast_convert_type(s & _KEY_MASK, jnp.float32)
           for s in sel]
    rec = [1.0 / jnp.maximum(d, 1e-10) for d in dst]
    norm = rec[0] + rec[1] + rec[2]
    w = [r / norm for r in rec]

    wt = (jnp.where(iota == idx[0], w[0], 0.0)
          + jnp.where(iota == idx[1], w[1], 0.0)
          + jnp.where(iota == idx[2], w[2], 0.0))

    interp = jnp.dot(p2_ref[0], wt, preferred_element_type=jnp.float32)
    xcat = jnp.concatenate([interp, p1_ref[0]], axis=0)
    y1 = jnp.dot(w1_ref[...], xcat,
                 preferred_element_type=jnp.float32) + b1_ref[...]
    y1_ref[0] = y1
    st_ref[:, 0:1] += jnp.sum(y1, axis=1, keepdims=True)
    st_ref[:, 1:2] += jnp.sum(y1 * y1, axis=1, keepdims=True)


def _pass_b(y1_ref, st1_ref, g1_ref, be1_ref, w2_ref, b2_ref,
            y2_ref, st2_ref, *, cnt):
    @pl.when(jnp.logical_and(pl.program_id(0) == 0, pl.program_id(1) == 0))
    def _init():
        st2_ref[...] = jnp.zeros_like(st2_ref)

    mean = st1_ref[:, 0:1] / cnt
    var = st1_ref[:, 1:2] / cnt - mean * mean
    inv = g1_ref[...] * jax.lax.rsqrt(var + 1e-5)
    h = jnp.maximum((y1_ref[0] - mean) * inv + be1_ref[...], 0.0)
    y2 = jnp.dot(w2_ref[...], h,
                 preferred_element_type=jnp.float32) + b2_ref[...]
    y2_ref[0] = y2
    st2_ref[:, 0:1] += jnp.sum(y2, axis=1, keepdims=True)
    st2_ref[:, 1:2] += jnp.sum(y2 * y2, axis=1, keepdims=True)


def _pass_c(y2_ref, st2_ref, g2_ref, be2_ref, out_ref, *, cnt):
    mean = st2_ref[:, 0:1] / cnt
    var = st2_ref[:, 1:2] / cnt - mean * mean
    inv = g2_ref[...] * jax.lax.rsqrt(var + 1e-5)
    out_ref[0] = jnp.maximum((y2_ref[0] - mean) * inv + be2_ref[...], 0.0)


@jax.jit
def kernel(xyz1, xyz2, points1, points2, W1, b1, g1, be1, W2, b2, g2, be2):
    b_, n, _ = xyz1.shape
    m = xyz2.shape[1]
    c1 = points1.shape[1]
    c2 = points2.shape[1]
    h1 = W1.shape[0]
    h2 = W2.shape[0]
    tn = 256 if n % 256 == 0 else 128
    nt = n // tn
    cnt = float(b_ * n)

    xyz1p = jnp.zeros((b_, 8, n), jnp.float32)
    xyz1p = xyz1p.at[:, 0:3, :].set(jnp.swapaxes(xyz1, 1, 2))

    col = lambda v: v.reshape(-1, 1)

    y1, st1 = pl.pallas_call(
        functools.partial(_pass_a, m=m, tn=tn),
        grid=(b_, nt),
        in_specs=[
            pl.BlockSpec((1, 8, tn), lambda b, i: (b, 0, i)),
            pl.BlockSpec((1, m, 3), lambda b, i: (b, 0, 0)),
            pl.BlockSpec((1, c2, m), lambda b, i: (b, 0, 0)),
            pl.BlockSpec((1, c1, tn), lambda b, i: (b, 0, i)),
            pl.BlockSpec((h1, c1 + c2), lambda b, i: (0, 0)),
            pl.BlockSpec((h1, 1), lambda b, i: (0, 0)),
        ],
        out_specs=[
            pl.BlockSpec((1, h1, tn), lambda b, i: (b, 0, i)),
            pl.BlockSpec((h1, 8), lambda b, i: (0, 0)),
        ],
        out_shape=[
            jax.ShapeDtypeStruct((b_, h1, n), jnp.float32),
            jax.ShapeDtypeStruct((h1, 8), jnp.float32),
        ],
    )(xyz1p, xyz2, points2, points1, W1, col(b1))

    y2, st2 = pl.pallas_call(
        functools.partial(_pass_b, cnt=cnt),
        grid=(b_, nt),
        in_specs=[
            pl.BlockSpec((1, h1, tn), lambda b, i: (b, 0, i)),
            pl.BlockSpec((h1, 8), lambda b, i: (0, 0)),
            pl.BlockSpec((h1, 1), lambda b, i: (0, 0)),
            pl.BlockSpec((h1, 1), lambda b, i: (0, 0)),
            pl.BlockSpec((h2, h1), lambda b, i: (0, 0)),
            pl.BlockSpec((h2, 1), lambda b, i: (0, 0)),
        ],
        out_specs=[
            pl.BlockSpec((1, h2, tn), lambda b, i: (b, 0, i)),
            pl.BlockSpec((h2, 8), lambda b, i: (0, 0)),
        ],
        out_shape=[
            jax.ShapeDtypeStruct((b_, h2, n), jnp.float32),
            jax.ShapeDtypeStruct((h2, 8), jnp.float32),
        ],
    )(y1, st1, col(g1), col(be1), W2, col(b2))

    out = pl.pallas_call(
        functools.partial(_pass_c, cnt=cnt),
        grid=(b_, nt),
        in_specs=[
            pl.BlockSpec((1, h2, tn), lambda b, i: (b, 0, i)),
            pl.BlockSpec((h2, 8), lambda b, i: (0, 0)),
            pl.BlockSpec((h2, 1), lambda b, i: (0, 0)),
            pl.BlockSpec((h2, 1), lambda b, i: (0, 0)),
        ],
        out_specs=pl.BlockSpec((1, h2, tn), lambda b, i: (b, 0, i)),
        out_shape=jax.ShapeDtypeStruct((b_, h2, n), jnp.float32),
    )(y2, st2, col(g2), col(be2))

    return out


# speedup vs baseline: 27.3689x; 27.3689x over previous
import functools

import jax
import jax.numpy as jnp
from jax.experimental import pallas as pl

_IDX_BITS = 12
_IDX_MASK = (1 << _IDX_BITS) - 1
_KEY_MASK = ~_IDX_MASK
_INF_KEY = 0x7FFFFFFF


def _pass_a(xyz1_ref, xyz2_ref, p2_ref, p1_ref, w1_ref, b1_ref,
            y1_ref, st_ref, *, m, tn):
    @pl.when(jnp.logical_and(pl.program_id(0) == 0, pl.program_id(1) == 0))
    def _init():
        st_ref[...] = jnp.zeros_like(st_ref)

    x = xyz1_ref[0]
    y = xyz2_ref[0]
    xb = x.astype(jnp.bfloat16).astype(jnp.float32)
    yb = y.astype(jnp.bfloat16).astype(jnp.float32)
    cross = ((yb[:, 0:1] * xb[0:1, :] + yb[:, 1:2] * xb[1:2, :])
             + yb[:, 2:3] * xb[2:3, :])
    sq1 = ((x[0:1, :] * x[0:1, :] + x[1:2, :] * x[1:2, :])
           + x[2:3, :] * x[2:3, :])
    sq2 = ((y[:, 0:1] * y[:, 0:1] + y[:, 1:2] * y[:, 1:2])
           + y[:, 2:3] * y[:, 2:3])
    d2 = (sq1 + sq2) - 2.0 * cross

    iota = jax.lax.broadcasted_iota(jnp.int32, (m, tn), 0)
    dst = []
    idx = []
    for _ in range(3):
        mn = jnp.min(d2, axis=0, keepdims=True)
        im = jnp.min(jnp.where(d2 == mn, iota, m), axis=0, keepdims=True)
        dst.append(mn)
        idx.append(im)
        d2 = jnp.where(iota == im, jnp.float32(3.4e38), d2)

    rec = [1.0 / jnp.maximum(d, 1e-10) for d in dst]
    norm = rec[0] + rec[1] + rec[2]
    w = [r / norm for r in rec]

    wt = (jnp.where(iota == idx[0], w[0], 0.0)
          + jnp.where(iota == idx[1], w[1], 0.0)
          + jnp.where(iota == idx[2], w[2], 0.0))

    interp = jnp.dot(p2_ref[0], wt, preferred_element_type=jnp.float32)
    xcat = jnp.concatenate([interp, p1_ref[0]], axis=0)
    y1 = jnp.dot(w1_ref[...], xcat,
                 preferred_element_type=jnp.float32) + b1_ref[...]
    y1_ref[0] = y1
    st_ref[:, 0:1] += jnp.sum(y1, axis=1, keepdims=True)
    st_ref[:, 1:2] += jnp.sum(y1 * y1, axis=1, keepdims=True)


def _pass_b(y1_ref, st1_ref, g1_ref, be1_ref, w2_ref, b2_ref,
            y2_ref, st2_ref, *, cnt):
    @pl.when(jnp.logical_and(pl.program_id(0) == 0, pl.program_id(1) == 0))
    def _init():
        st2_ref[...] = jnp.zeros_like(st2_ref)

    mean = st1_ref[:, 0:1] / cnt
    var = st1_ref[:, 1:2] / cnt - mean * mean
    inv = g1_ref[...] * jax.lax.rsqrt(var + 1e-5)
    h = jnp.maximum((y1_ref[0] - mean) * inv + be1_ref[...], 0.0)
    y2 = jnp.dot(w2_ref[...], h,
                 preferred_element_type=jnp.float32) + b2_ref[...]
    y2_ref[0] = y2
    st2_ref[:, 0:1] += jnp.sum(y2, axis=1, keepdims=True)
    st2_ref[:, 1:2] += jnp.sum(y2 * y2, axis=1, keepdims=True)


def _pass_c(y2_ref, st2_ref, g2_ref, be2_ref, out_ref, *, cnt):
    mean = st2_ref[:, 0:1] / cnt
    var = st2_ref[:, 1:2] / cnt - mean * mean
    inv = g2_ref[...] * jax.lax.rsqrt(var + 1e-5)
    out_ref[0] = jnp.maximum((y2_ref[0] - mean) * inv + be2_ref[...], 0.0)


@jax.jit
def kernel(xyz1, xyz2, points1, points2, W1, b1, g1, be1, W2, b2, g2, be2):
    b_, n, _ = xyz1.shape
    m = xyz2.shape[1]
    c1 = points1.shape[1]
    c2 = points2.shape[1]
    h1 = W1.shape[0]
    h2 = W2.shape[0]
    tn = 256 if n % 256 == 0 else 128
    nt = n // tn
    cnt = float(b_ * n)

    xyz1p = jnp.zeros((b_, 8, n), jnp.float32)
    xyz1p = xyz1p.at[:, 0:3, :].set(jnp.swapaxes(xyz1, 1, 2))

    col = lambda v: v.reshape(-1, 1)

    y1, st1 = pl.pallas_call(
        functools.partial(_pass_a, m=m, tn=tn),
        grid=(b_, nt),
        in_specs=[
            pl.BlockSpec((1, 8, tn), lambda b, i: (b, 0, i)),
            pl.BlockSpec((1, m, 3), lambda b, i: (b, 0, 0)),
            pl.BlockSpec((1, c2, m), lambda b, i: (b, 0, 0)),
            pl.BlockSpec((1, c1, tn), lambda b, i: (b, 0, i)),
            pl.BlockSpec((h1, c1 + c2), lambda b, i: (0, 0)),
            pl.BlockSpec((h1, 1), lambda b, i: (0, 0)),
        ],
        out_specs=[
            pl.BlockSpec((1, h1, tn), lambda b, i: (b, 0, i)),
            pl.BlockSpec((h1, 8), lambda b, i: (0, 0)),
        ],
        out_shape=[
            jax.ShapeDtypeStruct((b_, h1, n), jnp.float32),
            jax.ShapeDtypeStruct((h1, 8), jnp.float32),
        ],
    )(xyz1p, xyz2, points2, points1, W1, col(b1))

    y2, st2 = pl.pallas_call(
        functools.partial(_pass_b, cnt=cnt),
        grid=(b_, nt),
        in_specs=[
            pl.BlockSpec((1, h1, tn), lambda b, i: (b, 0, i)),
            pl.BlockSpec((h1, 8), lambda b, i: (0, 0)),
            pl.BlockSpec((h1, 1), lambda b, i: (0, 0)),
            pl.BlockSpec((h1, 1), lambda b, i: (0, 0)),
            pl.BlockSpec((h2, h1), lambda b, i: (0, 0)),
            pl.BlockSpec((h2, 1), lambda b, i: (0, 0)),
        ],
        out_specs=[
            pl.BlockSpec((1, h2, tn), lambda b, i: (b, 0, i)),
            pl.BlockSpec((h2, 8), lambda b, i: (0, 0)),
        ],
        out_shape=[
            jax.ShapeDtypeStruct((b_, h2, n), jnp.float32),
            jax.ShapeDtypeStruct((h2, 8), jnp.float32),
        ],
    )(y1, st1, col(g1), col(be1), W2, col(b2))

    out = pl.pallas_call(
        functools.partial(_pass_c, cnt=cnt),
        grid=(b_, nt),
        in_specs=[
            pl.BlockSpec((1, h2, tn), lambda b, i: (b, 0, i)),
            pl.BlockSpec((h2, 8), lambda b, i: (0, 0)),
            pl.BlockSpec((h2, 1), lambda b, i: (0, 0)),
            pl.BlockSpec((h2, 1), lambda b, i: (0, 0)),
        ],
        out_specs=pl.BlockSpec((1, h2, tn), lambda b, i: (b, 0, i)),
        out_shape=jax.ShapeDtypeStruct((b_, h2, n), jnp.float32),
    )(y2, st2, col(g2), col(be2))

    return out


# speedup vs baseline: 32.7328x; 1.1960x over previous
import functools

import jax
import jax.numpy as jnp
from jax.experimental import pallas as pl

_IDX_BITS = 12
_IDX_MASK = (1 << _IDX_BITS) - 1
_KEY_MASK = ~_IDX_MASK
_INF_KEY = 0x7FFFFFFF


def _pass_a(xyz1_ref, xyz2_ref, p2_ref, p1_ref, w1_ref, b1_ref,
            y1_ref, st_ref, *, m, tn):
    @pl.when(jnp.logical_and(pl.program_id(0) == 0, pl.program_id(1) == 0))
    def _init():
        st_ref[...] = jnp.zeros_like(st_ref)

    x = xyz1_ref[0]
    y = xyz2_ref[0]
    xb = x[0:3, :].astype(jnp.bfloat16)
    yb = y.astype(jnp.bfloat16)
    cross = jnp.dot(yb, xb, preferred_element_type=jnp.float32)
    sq1 = ((x[0:1, :] * x[0:1, :] + x[1:2, :] * x[1:2, :])
           + x[2:3, :] * x[2:3, :])
    sq2 = ((y[:, 0:1] * y[:, 0:1] + y[:, 1:2] * y[:, 1:2])
           + y[:, 2:3] * y[:, 2:3])
    d2 = (sq1 + sq2) - 2.0 * cross

    iota = jax.lax.broadcasted_iota(jnp.int32, (m, tn), 0)
    dst = []
    idx = []
    for k in range(3):
        mn = jnp.min(d2, axis=0, keepdims=True)
        im = jnp.min(jnp.where(d2 == mn, iota, m), axis=0, keepdims=True)
        dst.append(mn)
        idx.append(im)
        if k < 2:
            d2 = jnp.where(iota == im, jnp.float32(3.4e38), d2)

    rec = [1.0 / jnp.maximum(d, 1e-10) for d in dst]
    norm = rec[0] + rec[1] + rec[2]
    w = [r / norm for r in rec]

    wt = jnp.where(iota == idx[0], w[0],
                   jnp.where(iota == idx[1], w[1],
                             jnp.where(iota == idx[2], w[2], 0.0)))

    interp = jnp.dot(p2_ref[0], wt, preferred_element_type=jnp.float32)
    xcat = jnp.concatenate([interp, p1_ref[0]], axis=0)
    y1 = jnp.dot(w1_ref[...], xcat,
                 preferred_element_type=jnp.float32) + b1_ref[...]
    y1_ref[0] = y1
    st_ref[:, 0:1] += jnp.sum(y1, axis=1, keepdims=True)
    st_ref[:, 1:2] += jnp.sum(y1 * y1, axis=1, keepdims=True)


def _pass_b(y1_ref, st1_ref, g1_ref, be1_ref, w2_ref, b2_ref,
            y2_ref, st2_ref, *, cnt):
    @pl.when(jnp.logical_and(pl.program_id(0) == 0, pl.program_id(1) == 0))
    def _init():
        st2_ref[...] = jnp.zeros_like(st2_ref)

    mean = st1_ref[:, 0:1] / cnt
    var = st1_ref[:, 1:2] / cnt - mean * mean
    inv = g1_ref[...] * jax.lax.rsqrt(var + 1e-5)
    h = jnp.maximum((y1_ref[0] - mean) * inv + be1_ref[...], 0.0)
    y2 = jnp.dot(w2_ref[...], h,
                 preferred_element_type=jnp.float32) + b2_ref[...]
    y2_ref[0] = y2
    st2_ref[:, 0:1] += jnp.sum(y2, axis=1, keepdims=True)
    st2_ref[:, 1:2] += jnp.sum(y2 * y2, axis=1, keepdims=True)


def _pass_c(y2_ref, st2_ref, g2_ref, be2_ref, out_ref, *, cnt):
    mean = st2_ref[:, 0:1] / cnt
    var = st2_ref[:, 1:2] / cnt - mean * mean
    inv = g2_ref[...] * jax.lax.rsqrt(var + 1e-5)
    out_ref[0] = jnp.maximum((y2_ref[0] - mean) * inv + be2_ref[...], 0.0)


@jax.jit
def kernel(xyz1, xyz2, points1, points2, W1, b1, g1, be1, W2, b2, g2, be2):
    b_, n, _ = xyz1.shape
    m = xyz2.shape[1]
    c1 = points1.shape[1]
    c2 = points2.shape[1]
    h1 = W1.shape[0]
    h2 = W2.shape[0]
    tn = 256 if n % 256 == 0 else 128
    nt = n // tn
    cnt = float(b_ * n)

    xyz1p = jnp.zeros((b_, 8, n), jnp.float32)
    xyz1p = xyz1p.at[:, 0:3, :].set(jnp.swapaxes(xyz1, 1, 2))

    col = lambda v: v.reshape(-1, 1)

    y1, st1 = pl.pallas_call(
        functools.partial(_pass_a, m=m, tn=tn),
        grid=(b_, nt),
        in_specs=[
            pl.BlockSpec((1, 8, tn), lambda b, i: (b, 0, i)),
            pl.BlockSpec((1, m, 3), lambda b, i: (b, 0, 0)),
            pl.BlockSpec((1, c2, m), lambda b, i: (b, 0, 0)),
            pl.BlockSpec((1, c1, tn), lambda b, i: (b, 0, i)),
            pl.BlockSpec((h1, c1 + c2), lambda b, i: (0, 0)),
            pl.BlockSpec((h1, 1), lambda b, i: (0, 0)),
        ],
        out_specs=[
            pl.BlockSpec((1, h1, tn), lambda b, i: (b, 0, i)),
            pl.BlockSpec((h1, 8), lambda b, i: (0, 0)),
        ],
        out_shape=[
            jax.ShapeDtypeStruct((b_, h1, n), jnp.float32),
            jax.ShapeDtypeStruct((h1, 8), jnp.float32),
        ],
    )(xyz1p, xyz2, points2, points1, W1, col(b1))

    y2, st2 = pl.pallas_call(
        functools.partial(_pass_b, cnt=cnt),
        grid=(b_, nt),
        in_specs=[
            pl.BlockSpec((1, h1, tn), lambda b, i: (b, 0, i)),
            pl.BlockSpec((h1, 8), lambda b, i: (0, 0)),
            pl.BlockSpec((h1, 1), lambda b, i: (0, 0)),
            pl.BlockSpec((h1, 1), lambda b, i: (0, 0)),
            pl.BlockSpec((h2, h1), lambda b, i: (0, 0)),
            pl.BlockSpec((h2, 1), lambda b, i: (0, 0)),
        ],
        out_specs=[
            pl.BlockSpec((1, h2, tn), lambda b, i: (b, 0, i)),
            pl.BlockSpec((h2, 8), lambda b, i: (0, 0)),
        ],
        out_shape=[
            jax.ShapeDtypeStruct((b_, h2, n), jnp.float32),
            jax.ShapeDtypeStruct((h2, 8), jnp.float32),
        ],
    )(y1, st1, col(g1), col(be1), W2, col(b2))

    out = pl.pallas_call(
        functools.partial(_pass_c, cnt=cnt),
        grid=(b_, nt),
        in_specs=[
            pl.BlockSpec((1, h2, tn), lambda b, i: (b, 0, i)),
            pl.BlockSpec((h2, 8), lambda b, i: (0, 0)),
            pl.BlockSpec((h2, 1), lambda b, i: (0, 0)),
            pl.BlockSpec((h2, 1), lambda b, i: (0, 0)),
        ],
        out_specs=pl.BlockSpec((1, h2, tn), lambda b, i: (b, 0, i)),
        out_shape=jax.ShapeDtypeStruct((b_, h2, n), jnp.float32),
    )(y2, st2, col(g2), col(be2))

    return out
